# baseline (speedup 1.0000x reference)
import jax
import jax.numpy as jnp
from jax.experimental import pallas as pl


def _support_kernel(x_ref, w_ref, s_ref):
    s_ref[...] = jnp.dot(x_ref[...], w_ref[...],
                         preferred_element_type=jnp.float32)


def _gcn_kernel(adj_ref, s_ref, b_ref, o_ref):
    acc = jnp.dot(adj_ref[...], s_ref[...],
                  preferred_element_type=jnp.float32)
    o_ref[...] = jnp.maximum(acc + b_ref[...], 0.0)


def kernel(x, adj, W, b):
    n_rows, d_in = x.shape
    d_out = W.shape[1]
    n_cols = adj.shape[1]

    support = pl.pallas_call(
        _support_kernel,
        out_shape=jax.ShapeDtypeStruct((n_rows, d_out), jnp.float32),
    )(x, W)

    bm = 400
    b2 = b.reshape(1, d_out)
    out = pl.pallas_call(
        _gcn_kernel,
        grid=(pl.cdiv(n_rows, bm),),
        in_specs=[
            pl.BlockSpec((bm, n_cols), lambda m: (m, 0)),
            pl.BlockSpec((n_cols, d_out), lambda m: (0, 0)),
            pl.BlockSpec((1, d_out), lambda m: (0, 0)),
        ],
        out_specs=pl.BlockSpec((bm, d_out), lambda m: (m, 0)),
        out_shape=jax.ShapeDtypeStruct((n_rows, d_out), jnp.float32),
    )(adj, support, b2)
    return out


# speedup vs baseline: 1.0369x; 1.0369x over previous
import jax
import jax.numpy as jnp
from jax.experimental import pallas as pl
from jax.experimental.pallas import tpu as pltpu


def _gcn_kernel(x_ref, w_ref, b_ref, adj_ref, o_ref, s_ref):
    @pl.when(pl.program_id(0) == 0)
    def _():
        s_ref[...] = jnp.dot(x_ref[...], w_ref[...],
                             preferred_element_type=jnp.float32)

    acc = jnp.dot(adj_ref[...], s_ref[...],
                  preferred_element_type=jnp.float32)
    o_ref[...] = jnp.maximum(acc + b_ref[...], 0.0)


def kernel(x, adj, W, b):
    n_rows, d_in = x.shape
    d_out = W.shape[1]
    n_cols = adj.shape[1]

    bm = 200
    b2 = b.reshape(1, d_out)
    out = pl.pallas_call(
        _gcn_kernel,
        grid=(pl.cdiv(n_rows, bm),),
        in_specs=[
            pl.BlockSpec((n_rows, d_in), lambda m: (0, 0)),
            pl.BlockSpec((d_in, d_out), lambda m: (0, 0)),
            pl.BlockSpec((1, d_out), lambda m: (0, 0)),
            pl.BlockSpec((bm, n_cols), lambda m: (m, 0)),
        ],
        out_specs=pl.BlockSpec((bm, d_out), lambda m: (m, 0)),
        out_shape=jax.ShapeDtypeStruct((n_rows, d_out), jnp.float32),
        scratch_shapes=[pltpu.VMEM((n_cols, d_out), jnp.float32)],
    )(x, W, b2, adj)
    return out


# speedup vs baseline: 1.0459x; 1.0087x over previous
import jax
import jax.numpy as jnp
from jax.experimental import pallas as pl
from jax.experimental.pallas import tpu as pltpu


def _gcn_kernel(x_ref, w_ref, b_ref, adj_ref, o_ref, s_ref):
    @pl.when(pl.program_id(0) == 0)
    def _():
        s_ref[...] = jnp.dot(x_ref[...], w_ref[...],
                             preferred_element_type=jnp.float32)

    acc = jnp.dot(adj_ref[...], s_ref[...],
                  preferred_element_type=jnp.float32)
    o_ref[...] = jnp.maximum(acc + b_ref[...], 0.0)


def kernel(x, adj, W, b):
    n_rows, d_in = x.shape
    d_out = W.shape[1]
    n_cols = adj.shape[1]

    bm = 400
    b2 = b.reshape(1, d_out)
    out = pl.pallas_call(
        _gcn_kernel,
        grid=(pl.cdiv(n_rows, bm),),
        in_specs=[
            pl.BlockSpec((n_rows, d_in), lambda m: (0, 0)),
            pl.BlockSpec((d_in, d_out), lambda m: (0, 0)),
            pl.BlockSpec((1, d_out), lambda m: (0, 0)),
            pl.BlockSpec((bm, n_cols), lambda m: (m, 0)),
        ],
        out_specs=pl.BlockSpec((bm, d_out), lambda m: (m, 0)),
        out_shape=jax.ShapeDtypeStruct((n_rows, d_out), jnp.float32),
        scratch_shapes=[pltpu.VMEM((n_cols, d_out), jnp.float32)],
    )(x, W, b2, adj)
    return out
